# baseline (speedup 1.0000x reference)
import functools

import jax
import jax.numpy as jnp
from jax import lax
from jax.experimental import pallas as pl

_INTERPRET = False

_BN = 2048


def _fuse_kernel(nb_total, count, xyz1_ref, xyz2_ref, p1t_ref, p2t_ref,
                 w1t_ref, w2t_ref, bf_ref, xpre_ref, ssum_ref, ssq_ref):
    b = pl.program_id(0)
    nb = pl.program_id(1)
    x1 = xyz1_ref[0]
    x2 = xyz2_ref[0]
    dot = lax.dot_general(x1, x2, (((1,), (1,)), ((), ())),
                          preferred_element_type=jnp.float32)
    s1 = jnp.sum(x1 * x1, axis=1, keepdims=True)
    s2 = jnp.sum(x2 * x2, axis=1, keepdims=True)
    d = (-2.0 * dot + s1) + jnp.reshape(s2, (1, -1))
    t3 = jnp.max(d, axis=1, keepdims=True)
    for _ in range(2):
        t3 = jnp.max(jnp.where(d < t3, d, -jnp.inf), axis=1, keepdims=True)
    sel = d >= t3
    recip = jnp.where(sel, 1.0 / (d + 1e-8), 0.0)
    norm = jnp.sum(recip, axis=1, keepdims=True)
    w = recip / norm
    interp = lax.dot_general(w, p2t_ref[0], (((1,), (0,)), ((), ())),
                             preferred_element_type=jnp.float32)
    xpre = (lax.dot_general(p1t_ref[0], w1t_ref[...], (((1,), (0,)), ((), ())),
                            preferred_element_type=jnp.float32)
            + lax.dot_general(interp, w2t_ref[...], (((1,), (0,)), ((), ())),
                              preferred_element_type=jnp.float32)
            + bf_ref[...])
    xpre_ref[0] = xpre

    @pl.when(jnp.logical_and(b == 0, nb == 0))
    def _():
        ssum_ref[...] = jnp.zeros_like(ssum_ref)
        ssq_ref[...] = jnp.zeros_like(ssq_ref)

    ssum_ref[...] += jnp.sum(xpre, axis=0, keepdims=True)
    ssq_ref[...] += jnp.sum(xpre * xpre, axis=0, keepdims=True)


def _lap_kernel(count, k, xyzq_ref, xyza_ref, xpre_all_ref, xpre_blk_ref,
                ssum_ref, ssq_ref, g1_ref, b1_ref, wlut_ref, blu_ref,
                hpre_ref, ssum2_ref, ssq2_ref):
    b = pl.program_id(0)
    nb = pl.program_id(1)
    mu = ssum_ref[...] / count
    var = ssq_ref[...] / count - mu * mu
    inv = lax.rsqrt(var + 1e-5) * g1_ref[...]
    sh = b1_ref[...] - mu * inv
    xn_all = jnp.maximum(xpre_all_ref[0] * inv + sh, 0.0)
    xn_blk = jnp.maximum(xpre_blk_ref[0] * inv + sh, 0.0)
    xq = xyzq_ref[0]
    xa = xyza_ref[0]
    dot = lax.dot_general(xq, xa, (((1,), (1,)), ((), ())),
                          preferred_element_type=jnp.float32)
    sq = jnp.sum(xq * xq, axis=1, keepdims=True)
    sa = jnp.sum(xa * xa, axis=1, keepdims=True)
    m = (-2.0 * dot + jnp.reshape(sa, (1, -1))) + sq
    mn = jnp.min(m, axis=1, keepdims=True)
    for _ in range(k - 1):
        mn = jnp.min(jnp.where(m > mn, m, jnp.inf), axis=1, keepdims=True)
    maskf = (m <= mn).astype(jnp.float32)
    summed = lax.dot_general(maskf, xn_all, (((1,), (0,)), ((), ())),
                             preferred_element_type=jnp.float32)
    dx = summed - xn_blk
    hpre = jnp.maximum(
        lax.dot_general(dx, wlut_ref[...], (((1,), (0,)), ((), ())),
                        preferred_element_type=jnp.float32) + blu_ref[...],
        0.0)
    hpre_ref[0] = hpre

    @pl.when(jnp.logical_and(b == 0, nb == 0))
    def _():
        ssum2_ref[...] = jnp.zeros_like(ssum2_ref)
        ssq2_ref[...] = jnp.zeros_like(ssq2_ref)

    ssum2_ref[...] += jnp.sum(hpre, axis=0, keepdims=True)
    ssq2_ref[...] += jnp.sum(hpre * hpre, axis=0, keepdims=True)


def _final_kernel(count, xpre_ref, hpre_ref, ssum_ref, ssq_ref, g1_ref,
                  b1_ref, ssum2_ref, ssq2_ref, g2_ref, b2_ref, out_ref):
    mu1 = ssum_ref[...] / count
    var1 = ssq_ref[...] / count - mu1 * mu1
    inv1 = lax.rsqrt(var1 + 1e-5) * g1_ref[...]
    sh1 = b1_ref[...] - mu1 * inv1
    x = jnp.maximum(xpre_ref[0] * inv1 + sh1, 0.0)
    mu2 = ssum2_ref[...] / count
    var2 = ssq2_ref[...] / count - mu2 * mu2
    inv2 = lax.rsqrt(var2 + 1e-5) * g2_ref[...]
    sh2 = b2_ref[...] - mu2 * inv2
    h = hpre_ref[0] * inv2 + sh2
    out_ref[0] = jnp.transpose(x + h)


def kernel(xyz1, xyz2, points1, points2, W_fuse, b_fuse, gamma1, beta1,
           W_lu, b_lu, gamma2, beta2):
    B, N, _ = xyz1.shape
    S = xyz2.shape[1]
    C1 = points1.shape[1]
    C2 = points2.shape[1]
    Co = W_fuse.shape[0]
    bn = _BN
    nb_total = N // bn
    count = float(B * N)

    p1t = jnp.transpose(points1, (0, 2, 1))
    p2t = jnp.transpose(points2, (0, 2, 1))
    w1t = jnp.transpose(W_fuse[:, :C1])
    w2t = jnp.transpose(W_fuse[:, C1:])
    wlut = jnp.transpose(W_lu)
    row = lambda v: jnp.reshape(v, (1, -1))

    grid = (B, nb_total)
    stat_spec = pl.BlockSpec((1, Co), lambda b, n: (0, 0))
    full_spec = lambda r, c: pl.BlockSpec((1, r, c), lambda b, n: (b, 0, 0))
    blk_spec = lambda c: pl.BlockSpec((1, bn, c), lambda b, n: (b, n, 0))
    mat_spec = lambda r, c: pl.BlockSpec((r, c), lambda b, n: (0, 0))

    xpre, ssum, ssq = pl.pallas_call(
        functools.partial(_fuse_kernel, nb_total, count),
        grid=grid,
        in_specs=[blk_spec(3), full_spec(S, 3), blk_spec(C1), full_spec(S, C2),
                  mat_spec(C1, Co), mat_spec(C2, Co), stat_spec],
        out_specs=[blk_spec(Co), stat_spec, stat_spec],
        out_shape=[jax.ShapeDtypeStruct((B, N, Co), jnp.float32),
                   jax.ShapeDtypeStruct((1, Co), jnp.float32),
                   jax.ShapeDtypeStruct((1, Co), jnp.float32)],
        interpret=_INTERPRET,
    )(xyz1, xyz2, p1t, p2t, w1t, w2t, row(b_fuse))

    hpre, ssum2, ssq2 = pl.pallas_call(
        functools.partial(_lap_kernel, count, 16),
        grid=grid,
        in_specs=[blk_spec(3), full_spec(N, 3), full_spec(N, Co), blk_spec(Co),
                  stat_spec, stat_spec, stat_spec, stat_spec,
                  mat_spec(Co, Co), stat_spec],
        out_specs=[blk_spec(Co), stat_spec, stat_spec],
        out_shape=[jax.ShapeDtypeStruct((B, N, Co), jnp.float32),
                   jax.ShapeDtypeStruct((1, Co), jnp.float32),
                   jax.ShapeDtypeStruct((1, Co), jnp.float32)],
        interpret=_INTERPRET,
    )(xyz1, xyz1, xpre, xpre, ssum, ssq, row(gamma1), row(beta1),
      wlut, row(b_lu))

    out = pl.pallas_call(
        functools.partial(_final_kernel, count),
        grid=grid,
        in_specs=[blk_spec(Co), blk_spec(Co), stat_spec, stat_spec,
                  stat_spec, stat_spec, stat_spec, stat_spec,
                  stat_spec, stat_spec],
        out_specs=pl.BlockSpec((1, Co, bn), lambda b, n: (b, 0, n)),
        out_shape=jax.ShapeDtypeStruct((B, Co, N), jnp.float32),
        interpret=_INTERPRET,
    )(xpre, hpre, ssum, ssq, row(gamma1), row(beta1),
      ssum2, ssq2, row(gamma2), row(beta2))
    return out


# speedup vs baseline: 1.0382x; 1.0382x over previous
import functools

import jax
import jax.numpy as jnp
from jax import lax
from jax.experimental import pallas as pl

_INTERPRET = False

_BN = 1024


def _fuse_kernel(nb_total, count, xyz1_ref, xyz2_ref, p1t_ref, p2t_ref,
                 w1t_ref, w2t_ref, bf_ref, xpre_ref, ssum_ref, ssq_ref):
    b = pl.program_id(0)
    nb = pl.program_id(1)
    x1 = xyz1_ref[0]
    x2 = xyz2_ref[0]
    dot = lax.dot_general(x1, x2, (((1,), (1,)), ((), ())),
                          preferred_element_type=jnp.float32)
    s1 = jnp.sum(x1 * x1, axis=1, keepdims=True)
    s2 = jnp.sum(x2 * x2, axis=1, keepdims=True)
    d = (-2.0 * dot + s1) + jnp.reshape(s2, (1, -1))
    t3 = jnp.max(d, axis=1, keepdims=True)
    for _ in range(2):
        t3 = jnp.max(jnp.where(d < t3, d, -jnp.inf), axis=1, keepdims=True)
    sel = d >= t3
    recip = jnp.where(sel, 1.0 / (d + 1e-8), 0.0)
    norm = jnp.sum(recip, axis=1, keepdims=True)
    interp = lax.dot_general(recip, p2t_ref[0], (((1,), (0,)), ((), ())),
                             preferred_element_type=jnp.float32) / norm
    xpre = (lax.dot_general(p1t_ref[0], w1t_ref[...], (((1,), (0,)), ((), ())),
                            preferred_element_type=jnp.float32)
            + lax.dot_general(interp, w2t_ref[...], (((1,), (0,)), ((), ())),
                              preferred_element_type=jnp.float32)
            + bf_ref[...])
    xpre_ref[0] = xpre

    @pl.when(jnp.logical_and(b == 0, nb == 0))
    def _():
        ssum_ref[...] = jnp.zeros_like(ssum_ref)
        ssq_ref[...] = jnp.zeros_like(ssq_ref)

    ssum_ref[...] += jnp.sum(xpre, axis=0, keepdims=True)
    ssq_ref[...] += jnp.sum(xpre * xpre, axis=0, keepdims=True)


def _lap_kernel(count, k, xyzq_ref, xyza_ref, xpre_all_ref, xpre_blk_ref,
                ssum_ref, ssq_ref, g1_ref, b1_ref, wlut_ref, blu_ref,
                hpre_ref, ssum2_ref, ssq2_ref):
    b = pl.program_id(0)
    nb = pl.program_id(1)
    mu = ssum_ref[...] / count
    var = ssq_ref[...] / count - mu * mu
    inv = lax.rsqrt(var + 1e-5) * g1_ref[...]
    sh = b1_ref[...] - mu * inv
    xn_all = jnp.maximum(xpre_all_ref[0] * inv + sh, 0.0)
    xn_blk = jnp.maximum(xpre_blk_ref[0] * inv + sh, 0.0)
    xq = xyzq_ref[0]
    xa = xyza_ref[0]
    dot = lax.dot_general(xq, xa, (((1,), (1,)), ((), ())),
                          preferred_element_type=jnp.float32)
    sq = jnp.sum(xq * xq, axis=1, keepdims=True)
    sa = jnp.sum(xa * xa, axis=1, keepdims=True)
    m = (-2.0 * dot + jnp.reshape(sa, (1, -1))) + sq
    mn = jnp.min(m, axis=1, keepdims=True)
    for _ in range(k - 1):
        mn = jnp.min(jnp.where(m > mn, m, jnp.inf), axis=1, keepdims=True)
    maskf = (m <= mn).astype(jnp.float32)
    summed = lax.dot_general(maskf, xn_all, (((1,), (0,)), ((), ())),
                             preferred_element_type=jnp.float32)
    dx = summed - xn_blk
    hpre = jnp.maximum(
        lax.dot_general(dx, wlut_ref[...], (((1,), (0,)), ((), ())),
                        preferred_element_type=jnp.float32) + blu_ref[...],
        0.0)
    hpre_ref[0] = hpre

    @pl.when(jnp.logical_and(b == 0, nb == 0))
    def _():
        ssum2_ref[...] = jnp.zeros_like(ssum2_ref)
        ssq2_ref[...] = jnp.zeros_like(ssq2_ref)

    ssum2_ref[...] += jnp.sum(hpre, axis=0, keepdims=True)
    ssq2_ref[...] += jnp.sum(hpre * hpre, axis=0, keepdims=True)


def _final_kernel(count, xpre_ref, hpre_ref, ssum_ref, ssq_ref, g1_ref,
                  b1_ref, ssum2_ref, ssq2_ref, g2_ref, b2_ref, out_ref):
    mu1 = ssum_ref[...] / count
    var1 = ssq_ref[...] / count - mu1 * mu1
    inv1 = lax.rsqrt(var1 + 1e-5) * g1_ref[...]
    sh1 = b1_ref[...] - mu1 * inv1
    x = jnp.maximum(xpre_ref[0] * inv1 + sh1, 0.0)
    mu2 = ssum2_ref[...] / count
    var2 = ssq2_ref[...] / count - mu2 * mu2
    inv2 = lax.rsqrt(var2 + 1e-5) * g2_ref[...]
    sh2 = b2_ref[...] - mu2 * inv2
    h = hpre_ref[0] * inv2 + sh2
    out_ref[0] = jnp.transpose(x + h)


def kernel(xyz1, xyz2, points1, points2, W_fuse, b_fuse, gamma1, beta1,
           W_lu, b_lu, gamma2, beta2):
    B, N, _ = xyz1.shape
    S = xyz2.shape[1]
    C1 = points1.shape[1]
    C2 = points2.shape[1]
    Co = W_fuse.shape[0]
    bn = _BN
    nb_total = N // bn
    count = float(B * N)

    p1t = jnp.transpose(points1, (0, 2, 1))
    p2t = jnp.transpose(points2, (0, 2, 1))
    w1t = jnp.transpose(W_fuse[:, :C1])
    w2t = jnp.transpose(W_fuse[:, C1:])
    wlut = jnp.transpose(W_lu)
    row = lambda v: jnp.reshape(v, (1, -1))

    grid = (B, nb_total)
    stat_spec = pl.BlockSpec((1, Co), lambda b, n: (0, 0))
    full_spec = lambda r, c: pl.BlockSpec((1, r, c), lambda b, n: (b, 0, 0))
    blk_spec = lambda c: pl.BlockSpec((1, bn, c), lambda b, n: (b, n, 0))
    mat_spec = lambda r, c: pl.BlockSpec((r, c), lambda b, n: (0, 0))

    xpre, ssum, ssq = pl.pallas_call(
        functools.partial(_fuse_kernel, nb_total, count),
        grid=grid,
        in_specs=[blk_spec(3), full_spec(S, 3), blk_spec(C1), full_spec(S, C2),
                  mat_spec(C1, Co), mat_spec(C2, Co), stat_spec],
        out_specs=[blk_spec(Co), stat_spec, stat_spec],
        out_shape=[jax.ShapeDtypeStruct((B, N, Co), jnp.float32),
                   jax.ShapeDtypeStruct((1, Co), jnp.float32),
                   jax.ShapeDtypeStruct((1, Co), jnp.float32)],
        interpret=_INTERPRET,
    )(xyz1, xyz2, p1t, p2t, w1t, w2t, row(b_fuse))

    hpre, ssum2, ssq2 = pl.pallas_call(
        functools.partial(_lap_kernel, count, 16),
        grid=grid,
        in_specs=[blk_spec(3), full_spec(N, 3), full_spec(N, Co), blk_spec(Co),
                  stat_spec, stat_spec, stat_spec, stat_spec,
                  mat_spec(Co, Co), stat_spec],
        out_specs=[blk_spec(Co), stat_spec, stat_spec],
        out_shape=[jax.ShapeDtypeStruct((B, N, Co), jnp.float32),
                   jax.ShapeDtypeStruct((1, Co), jnp.float32),
                   jax.ShapeDtypeStruct((1, Co), jnp.float32)],
        interpret=_INTERPRET,
    )(xyz1, xyz1, xpre, xpre, ssum, ssq, row(gamma1), row(beta1),
      wlut, row(b_lu))

    out = pl.pallas_call(
        functools.partial(_final_kernel, count),
        grid=grid,
        in_specs=[blk_spec(Co), blk_spec(Co), stat_spec, stat_spec,
                  stat_spec, stat_spec, stat_spec, stat_spec,
                  stat_spec, stat_spec],
        out_specs=pl.BlockSpec((1, Co, bn), lambda b, n: (b, 0, n)),
        out_shape=jax.ShapeDtypeStruct((B, Co, N), jnp.float32),
        interpret=_INTERPRET,
    )(xpre, hpre, ssum, ssq, row(gamma1), row(beta1),
      ssum2, ssq2, row(gamma2), row(beta2))
    return out


# speedup vs baseline: 1.1394x; 1.0975x over previous
import functools

import jax
import jax.numpy as jnp
from jax import lax
from jax.experimental import pallas as pl

_INTERPRET = False

_BN = 1024


def _fuse_kernel(nb_total, count, xyz1_ref, xyz2_ref, p1t_ref, p2t_ref,
                 w1t_ref, w2t_ref, bf_ref, xpre_ref, ssum_ref, ssq_ref):
    b = pl.program_id(0)
    nb = pl.program_id(1)
    x1 = xyz1_ref[0]
    x2 = xyz2_ref[0]
    dot = lax.dot_general(x1, x2, (((1,), (1,)), ((), ())),
                          preferred_element_type=jnp.float32)
    s1 = jnp.sum(x1 * x1, axis=1, keepdims=True)
    s2 = jnp.sum(x2 * x2, axis=1, keepdims=True)
    d = (-2.0 * dot + s1) + jnp.reshape(s2, (1, -1))
    t3 = jnp.max(d, axis=1, keepdims=True)
    for _ in range(2):
        t3 = jnp.max(jnp.where(d < t3, d, -jnp.inf), axis=1, keepdims=True)
    sel = d >= t3
    recip = jnp.where(sel, 1.0 / (d + 1e-8), 0.0)
    norm = jnp.sum(recip, axis=1, keepdims=True)
    interp = lax.dot_general(recip, p2t_ref[0], (((1,), (0,)), ((), ())),
                             preferred_element_type=jnp.float32) / norm
    xpre = (lax.dot_general(p1t_ref[0], w1t_ref[...], (((1,), (0,)), ((), ())),
                            preferred_element_type=jnp.float32)
            + lax.dot_general(interp, w2t_ref[...], (((1,), (0,)), ((), ())),
                              preferred_element_type=jnp.float32)
            + bf_ref[...])
    xpre_ref[0] = xpre

    @pl.when(jnp.logical_and(b == 0, nb == 0))
    def _():
        ssum_ref[...] = jnp.zeros_like(ssum_ref)
        ssq_ref[...] = jnp.zeros_like(ssq_ref)

    ssum_ref[...] += jnp.sum(xpre, axis=0, keepdims=True)
    ssq_ref[...] += jnp.sum(xpre * xpre, axis=0, keepdims=True)


def _lap_kernel(count, k, xyzq_ref, xyza_ref, xpre_all_ref, xpre_blk_ref,
                ssum_ref, ssq_ref, g1_ref, b1_ref, wlut_ref, blu_ref,
                hpre_ref, ssum2_ref, ssq2_ref):
    b = pl.program_id(0)
    nb = pl.program_id(1)
    mu = ssum_ref[...] / count
    var = ssq_ref[...] / count - mu * mu
    inv = lax.rsqrt(var + 1e-5) * g1_ref[...]
    sh = b1_ref[...] - mu * inv
    xn_all = jnp.maximum(xpre_all_ref[0] * inv + sh, 0.0)
    xn_blk = jnp.maximum(xpre_blk_ref[0] * inv + sh, 0.0)
    xq = xyzq_ref[0]
    xa = xyza_ref[0]
    dot = lax.dot_general(xq, xa, (((1,), (1,)), ((), ())),
                          preferred_element_type=jnp.float32)
    sq = jnp.sum(xq * xq, axis=1, keepdims=True)
    sa = jnp.sum(xa * xa, axis=1, keepdims=True)
    m = (-2.0 * dot + jnp.reshape(sa, (1, -1))) + sq
    def _two_smallest(mm, thresh):
        a1 = a2 = None
        for c in range(mm.shape[1] // 128):
            v = mm[:, c * 128:(c + 1) * 128]
            if thresh is not None:
                v = jnp.where(v > thresh, v, jnp.inf)
            if a1 is None:
                a1, a2 = v, jnp.full_like(v, jnp.inf)
            else:
                t = jnp.maximum(a1, v)
                a1 = jnp.minimum(a1, v)
                a2 = jnp.minimum(a2, t)
        cat = jnp.concatenate([a1, a2], axis=1)
        t1 = jnp.min(cat, axis=1, keepdims=True)
        t2 = jnp.min(jnp.where(cat > t1, cat, jnp.inf), axis=1,
                     keepdims=True)
        return t2

    mn = _two_smallest(m, None)
    for _ in range(k // 2 - 1):
        mn = _two_smallest(m, mn)
    maskf = (m <= mn).astype(jnp.float32)
    summed = lax.dot_general(maskf, xn_all, (((1,), (0,)), ((), ())),
                             preferred_element_type=jnp.float32)
    dx = summed - xn_blk
    hpre = jnp.maximum(
        lax.dot_general(dx, wlut_ref[...], (((1,), (0,)), ((), ())),
                        preferred_element_type=jnp.float32) + blu_ref[...],
        0.0)
    hpre_ref[0] = hpre

    @pl.when(jnp.logical_and(b == 0, nb == 0))
    def _():
        ssum2_ref[...] = jnp.zeros_like(ssum2_ref)
        ssq2_ref[...] = jnp.zeros_like(ssq2_ref)

    ssum2_ref[...] += jnp.sum(hpre, axis=0, keepdims=True)
    ssq2_ref[...] += jnp.sum(hpre * hpre, axis=0, keepdims=True)


def _final_kernel(count, xpre_ref, hpre_ref, ssum_ref, ssq_ref, g1_ref,
                  b1_ref, ssum2_ref, ssq2_ref, g2_ref, b2_ref, out_ref):
    mu1 = ssum_ref[...] / count
    var1 = ssq_ref[...] / count - mu1 * mu1
    inv1 = lax.rsqrt(var1 + 1e-5) * g1_ref[...]
    sh1 = b1_ref[...] - mu1 * inv1
    x = jnp.maximum(xpre_ref[0] * inv1 + sh1, 0.0)
    mu2 = ssum2_ref[...] / count
    var2 = ssq2_ref[...] / count - mu2 * mu2
    inv2 = lax.rsqrt(var2 + 1e-5) * g2_ref[...]
    sh2 = b2_ref[...] - mu2 * inv2
    h = hpre_ref[0] * inv2 + sh2
    out_ref[0] = jnp.transpose(x + h)


def kernel(xyz1, xyz2, points1, points2, W_fuse, b_fuse, gamma1, beta1,
           W_lu, b_lu, gamma2, beta2):
    B, N, _ = xyz1.shape
    S = xyz2.shape[1]
    C1 = points1.shape[1]
    C2 = points2.shape[1]
    Co = W_fuse.shape[0]
    bn = _BN
    nb_total = N // bn
    count = float(B * N)

    p1t = jnp.transpose(points1, (0, 2, 1))
    p2t = jnp.transpose(points2, (0, 2, 1))
    w1t = jnp.transpose(W_fuse[:, :C1])
    w2t = jnp.transpose(W_fuse[:, C1:])
    wlut = jnp.transpose(W_lu)
    row = lambda v: jnp.reshape(v, (1, -1))

    grid = (B, nb_total)
    stat_spec = pl.BlockSpec((1, Co), lambda b, n: (0, 0))
    full_spec = lambda r, c: pl.BlockSpec((1, r, c), lambda b, n: (b, 0, 0))
    blk_spec = lambda c: pl.BlockSpec((1, bn, c), lambda b, n: (b, n, 0))
    mat_spec = lambda r, c: pl.BlockSpec((r, c), lambda b, n: (0, 0))

    xpre, ssum, ssq = pl.pallas_call(
        functools.partial(_fuse_kernel, nb_total, count),
        grid=grid,
        in_specs=[blk_spec(3), full_spec(S, 3), blk_spec(C1), full_spec(S, C2),
                  mat_spec(C1, Co), mat_spec(C2, Co), stat_spec],
        out_specs=[blk_spec(Co), stat_spec, stat_spec],
        out_shape=[jax.ShapeDtypeStruct((B, N, Co), jnp.float32),
                   jax.ShapeDtypeStruct((1, Co), jnp.float32),
                   jax.ShapeDtypeStruct((1, Co), jnp.float32)],
        interpret=_INTERPRET,
    )(xyz1, xyz2, p1t, p2t, w1t, w2t, row(b_fuse))

    hpre, ssum2, ssq2 = pl.pallas_call(
        functools.partial(_lap_kernel, count, 16),
        grid=grid,
        in_specs=[blk_spec(3), full_spec(N, 3), full_spec(N, Co), blk_spec(Co),
                  stat_spec, stat_spec, stat_spec, stat_spec,
                  mat_spec(Co, Co), stat_spec],
        out_specs=[blk_spec(Co), stat_spec, stat_spec],
        out_shape=[jax.ShapeDtypeStruct((B, N, Co), jnp.float32),
                   jax.ShapeDtypeStruct((1, Co), jnp.float32),
                   jax.ShapeDtypeStruct((1, Co), jnp.float32)],
        interpret=_INTERPRET,
    )(xyz1, xyz1, xpre, xpre, ssum, ssq, row(gamma1), row(beta1),
      wlut, row(b_lu))

    out = pl.pallas_call(
        functools.partial(_final_kernel, count),
        grid=grid,
        in_specs=[blk_spec(Co), blk_spec(Co), stat_spec, stat_spec,
                  stat_spec, stat_spec, stat_spec, stat_spec,
                  stat_spec, stat_spec],
        out_specs=pl.BlockSpec((1, Co, bn), lambda b, n: (b, 0, n)),
        out_shape=jax.ShapeDtypeStruct((B, Co, N), jnp.float32),
        interpret=_INTERPRET,
    )(xpre, hpre, ssum, ssq, row(gamma1), row(beta1),
      ssum2, ssq2, row(gamma2), row(beta2))
    return out


# speedup vs baseline: 1.1404x; 1.0009x over previous
import functools

import jax
import jax.numpy as jnp
from jax import lax
from jax.experimental import pallas as pl

_INTERPRET = False

_BN = 1024


def _fuse_kernel(nb_total, count, xyz1_ref, xyz2_ref, p1t_ref, p2t_ref,
                 w1t_ref, w2t_ref, bf_ref, xpre_ref, ssum_ref, ssq_ref):
    b = pl.program_id(0)
    nb = pl.program_id(1)
    x1 = xyz1_ref[0]
    x2 = xyz2_ref[0]
    dot = lax.dot_general(x1, x2, (((1,), (1,)), ((), ())),
                          preferred_element_type=jnp.float32)
    s1 = jnp.sum(x1 * x1, axis=1, keepdims=True)
    s2 = jnp.sum(x2 * x2, axis=1, keepdims=True)
    d = (-2.0 * dot + s1) + jnp.reshape(s2, (1, -1))
    b1 = b2 = None
    for c in range(d.shape[1] // 128):
        v = d[:, c * 128:(c + 1) * 128]
        if b1 is None:
            b1, b2 = v, jnp.full_like(v, -jnp.inf)
        else:
            t = jnp.minimum(b1, v)
            b1 = jnp.maximum(b1, v)
            b2 = jnp.maximum(b2, t)
    cat = jnp.concatenate([b1, b2], axis=1)
    t1 = jnp.max(cat, axis=1, keepdims=True)
    t2 = jnp.max(jnp.where(cat < t1, cat, -jnp.inf), axis=1, keepdims=True)
    t3 = jnp.max(jnp.where(d < t2, d, -jnp.inf), axis=1, keepdims=True)
    sel = d >= t3
    recip = jnp.where(sel, 1.0 / (d + 1e-8), 0.0)
    norm = jnp.sum(recip, axis=1, keepdims=True)
    interp = lax.dot_general(recip, p2t_ref[0], (((1,), (0,)), ((), ())),
                             preferred_element_type=jnp.float32) / norm
    xpre = (lax.dot_general(p1t_ref[0], w1t_ref[...], (((1,), (0,)), ((), ())),
                            preferred_element_type=jnp.float32)
            + lax.dot_general(interp, w2t_ref[...], (((1,), (0,)), ((), ())),
                              preferred_element_type=jnp.float32)
            + bf_ref[...])
    xpre_ref[0] = xpre

    @pl.when(jnp.logical_and(b == 0, nb == 0))
    def _():
        ssum_ref[...] = jnp.zeros_like(ssum_ref)
        ssq_ref[...] = jnp.zeros_like(ssq_ref)

    ssum_ref[...] += jnp.sum(xpre, axis=0, keepdims=True)
    ssq_ref[...] += jnp.sum(xpre * xpre, axis=0, keepdims=True)


def _lap_kernel(count, k, xyzq_ref, xyza_ref, xpre_all_ref, xpre_blk_ref,
                ssum_ref, ssq_ref, g1_ref, b1_ref, wlut_ref, blu_ref,
                hpre_ref, ssum2_ref, ssq2_ref):
    b = pl.program_id(0)
    nb = pl.program_id(1)
    mu = ssum_ref[...] / count
    var = ssq_ref[...] / count - mu * mu
    inv = lax.rsqrt(var + 1e-5) * g1_ref[...]
    sh = b1_ref[...] - mu * inv
    xn_all = jnp.maximum(xpre_all_ref[0] * inv + sh, 0.0)
    xn_blk = jnp.maximum(xpre_blk_ref[0] * inv + sh, 0.0)
    xq = xyzq_ref[0]
    xa = xyza_ref[0]
    dot = lax.dot_general(xq, xa, (((1,), (1,)), ((), ())),
                          preferred_element_type=jnp.float32)
    sq = jnp.sum(xq * xq, axis=1, keepdims=True)
    sa = jnp.sum(xa * xa, axis=1, keepdims=True)
    m = (-2.0 * dot + jnp.reshape(sa, (1, -1))) + sq
    def _two_smallest(mm, thresh):
        a1 = a2 = None
        for c in range(mm.shape[1] // 128):
            v = mm[:, c * 128:(c + 1) * 128]
            if thresh is not None:
                v = jnp.where(v > thresh, v, jnp.inf)
            if a1 is None:
                a1, a2 = v, jnp.full_like(v, jnp.inf)
            else:
                t = jnp.maximum(a1, v)
                a1 = jnp.minimum(a1, v)
                a2 = jnp.minimum(a2, t)
        cat = jnp.concatenate([a1, a2], axis=1)
        t1 = jnp.min(cat, axis=1, keepdims=True)
        t2 = jnp.min(jnp.where(cat > t1, cat, jnp.inf), axis=1,
                     keepdims=True)
        return t2

    mn = _two_smallest(m, None)
    for _ in range(k // 2 - 1):
        mn = _two_smallest(m, mn)
    maskf = (m <= mn).astype(jnp.float32)
    summed = lax.dot_general(maskf, xn_all, (((1,), (0,)), ((), ())),
                             preferred_element_type=jnp.float32)
    dx = summed - xn_blk
    hpre = jnp.maximum(
        lax.dot_general(dx, wlut_ref[...], (((1,), (0,)), ((), ())),
                        preferred_element_type=jnp.float32) + blu_ref[...],
        0.0)
    hpre_ref[0] = hpre

    @pl.when(jnp.logical_and(b == 0, nb == 0))
    def _():
        ssum2_ref[...] = jnp.zeros_like(ssum2_ref)
        ssq2_ref[...] = jnp.zeros_like(ssq2_ref)

    ssum2_ref[...] += jnp.sum(hpre, axis=0, keepdims=True)
    ssq2_ref[...] += jnp.sum(hpre * hpre, axis=0, keepdims=True)


def _final_kernel(count, xpre_ref, hpre_ref, ssum_ref, ssq_ref, g1_ref,
                  b1_ref, ssum2_ref, ssq2_ref, g2_ref, b2_ref, out_ref):
    mu1 = ssum_ref[...] / count
    var1 = ssq_ref[...] / count - mu1 * mu1
    inv1 = lax.rsqrt(var1 + 1e-5) * g1_ref[...]
    sh1 = b1_ref[...] - mu1 * inv1
    x = jnp.maximum(xpre_ref[0] * inv1 + sh1, 0.0)
    mu2 = ssum2_ref[...] / count
    var2 = ssq2_ref[...] / count - mu2 * mu2
    inv2 = lax.rsqrt(var2 + 1e-5) * g2_ref[...]
    sh2 = b2_ref[...] - mu2 * inv2
    h = hpre_ref[0] * inv2 + sh2
    out_ref[0] = jnp.transpose(x + h)


def kernel(xyz1, xyz2, points1, points2, W_fuse, b_fuse, gamma1, beta1,
           W_lu, b_lu, gamma2, beta2):
    B, N, _ = xyz1.shape
    S = xyz2.shape[1]
    C1 = points1.shape[1]
    C2 = points2.shape[1]
    Co = W_fuse.shape[0]
    bn = _BN
    nb_total = N // bn
    count = float(B * N)

    p1t = jnp.transpose(points1, (0, 2, 1))
    p2t = jnp.transpose(points2, (0, 2, 1))
    w1t = jnp.transpose(W_fuse[:, :C1])
    w2t = jnp.transpose(W_fuse[:, C1:])
    wlut = jnp.transpose(W_lu)
    row = lambda v: jnp.reshape(v, (1, -1))

    grid = (B, nb_total)
    stat_spec = pl.BlockSpec((1, Co), lambda b, n: (0, 0))
    full_spec = lambda r, c: pl.BlockSpec((1, r, c), lambda b, n: (b, 0, 0))
    blk_spec = lambda c: pl.BlockSpec((1, bn, c), lambda b, n: (b, n, 0))
    mat_spec = lambda r, c: pl.BlockSpec((r, c), lambda b, n: (0, 0))

    xpre, ssum, ssq = pl.pallas_call(
        functools.partial(_fuse_kernel, nb_total, count),
        grid=grid,
        in_specs=[blk_spec(3), full_spec(S, 3), blk_spec(C1), full_spec(S, C2),
                  mat_spec(C1, Co), mat_spec(C2, Co), stat_spec],
        out_specs=[blk_spec(Co), stat_spec, stat_spec],
        out_shape=[jax.ShapeDtypeStruct((B, N, Co), jnp.float32),
                   jax.ShapeDtypeStruct((1, Co), jnp.float32),
                   jax.ShapeDtypeStruct((1, Co), jnp.float32)],
        interpret=_INTERPRET,
    )(xyz1, xyz2, p1t, p2t, w1t, w2t, row(b_fuse))

    hpre, ssum2, ssq2 = pl.pallas_call(
        functools.partial(_lap_kernel, count, 16),
        grid=grid,
        in_specs=[blk_spec(3), full_spec(N, 3), full_spec(N, Co), blk_spec(Co),
                  stat_spec, stat_spec, stat_spec, stat_spec,
                  mat_spec(Co, Co), stat_spec],
        out_specs=[blk_spec(Co), stat_spec, stat_spec],
        out_shape=[jax.ShapeDtypeStruct((B, N, Co), jnp.float32),
                   jax.ShapeDtypeStruct((1, Co), jnp.float32),
                   jax.ShapeDtypeStruct((1, Co), jnp.float32)],
        interpret=_INTERPRET,
    )(xyz1, xyz1, xpre, xpre, ssum, ssq, row(gamma1), row(beta1),
      wlut, row(b_lu))

    out = pl.pallas_call(
        functools.partial(_final_kernel, count),
        grid=grid,
        in_specs=[blk_spec(Co), blk_spec(Co), stat_spec, stat_spec,
                  stat_spec, stat_spec, stat_spec, stat_spec,
                  stat_spec, stat_spec],
        out_specs=pl.BlockSpec((1, Co, bn), lambda b, n: (b, 0, n)),
        out_shape=jax.ShapeDtypeStruct((B, Co, N), jnp.float32),
        interpret=_INTERPRET,
    )(xpre, hpre, ssum, ssq, row(gamma1), row(beta1),
      ssum2, ssq2, row(gamma2), row(beta2))
    return out


# speedup vs baseline: 1.1773x; 1.0324x over previous
import functools

import jax
import jax.numpy as jnp
from jax import lax
from jax.experimental import pallas as pl

_INTERPRET = False

_BN = 2048


def _fuse_kernel(nb_total, count, xyz1_ref, xyz2_ref, p1t_ref, p2t_ref,
                 w1t_ref, w2t_ref, bf_ref, xpre_ref, ssum_ref, ssq_ref):
    b = pl.program_id(0)
    nb = pl.program_id(1)
    x1 = xyz1_ref[0]
    x2 = xyz2_ref[0]
    dot = lax.dot_general(x1, x2, (((1,), (1,)), ((), ())),
                          preferred_element_type=jnp.float32)
    s1 = jnp.sum(x1 * x1, axis=1, keepdims=True)
    s2 = jnp.sum(x2 * x2, axis=1, keepdims=True)
    d = (-2.0 * dot + s1) + jnp.reshape(s2, (1, -1))
    b1 = b2 = None
    for c in range(d.shape[1] // 128):
        v = d[:, c * 128:(c + 1) * 128]
        if b1 is None:
            b1, b2 = v, jnp.full_like(v, -jnp.inf)
        else:
            t = jnp.minimum(b1, v)
            b1 = jnp.maximum(b1, v)
            b2 = jnp.maximum(b2, t)
    cat = jnp.concatenate([b1, b2], axis=1)
    t1 = jnp.max(cat, axis=1, keepdims=True)
    t2 = jnp.max(jnp.where(cat < t1, cat, -jnp.inf), axis=1, keepdims=True)
    t3 = jnp.max(jnp.where(d < t2, d, -jnp.inf), axis=1, keepdims=True)
    sel = d >= t3
    recip = jnp.where(sel, 1.0 / (d + 1e-8), 0.0)
    norm = jnp.sum(recip, axis=1, keepdims=True)
    interp = lax.dot_general(recip, p2t_ref[0], (((1,), (0,)), ((), ())),
                             preferred_element_type=jnp.float32) / norm
    xpre = (lax.dot_general(p1t_ref[0], w1t_ref[...], (((1,), (0,)), ((), ())),
                            preferred_element_type=jnp.float32)
            + lax.dot_general(interp, w2t_ref[...], (((1,), (0,)), ((), ())),
                              preferred_element_type=jnp.float32)
            + bf_ref[...])
    xpre_ref[0] = xpre

    @pl.when(jnp.logical_and(b == 0, nb == 0))
    def _():
        ssum_ref[...] = jnp.zeros_like(ssum_ref)
        ssq_ref[...] = jnp.zeros_like(ssq_ref)

    ssum_ref[...] += jnp.sum(xpre, axis=0, keepdims=True)
    ssq_ref[...] += jnp.sum(xpre * xpre, axis=0, keepdims=True)


def _lap_kernel(count, k, xyzq_ref, xyza_ref, xpre_all_ref, xpre_blk_ref,
                ssum_ref, ssq_ref, g1_ref, b1_ref, wlut_ref, blu_ref,
                hpre_ref, ssum2_ref, ssq2_ref):
    b = pl.program_id(0)
    nb = pl.program_id(1)
    mu = ssum_ref[...] / count
    var = ssq_ref[...] / count - mu * mu
    inv = lax.rsqrt(var + 1e-5) * g1_ref[...]
    sh = b1_ref[...] - mu * inv
    xn_all = jnp.maximum(xpre_all_ref[0] * inv + sh, 0.0)
    xn_blk = jnp.maximum(xpre_blk_ref[0] * inv + sh, 0.0)
    xq = xyzq_ref[0]
    xa = xyza_ref[0]
    dot = lax.dot_general(xq, xa, (((1,), (1,)), ((), ())),
                          preferred_element_type=jnp.float32)
    sq = jnp.sum(xq * xq, axis=1, keepdims=True)
    sa = jnp.sum(xa * xa, axis=1, keepdims=True)
    m = (-2.0 * dot + jnp.reshape(sa, (1, -1))) + sq
    def _two_smallest(mm, thresh):
        a1 = a2 = None
        for c in range(mm.shape[1] // 128):
            v = mm[:, c * 128:(c + 1) * 128]
            if thresh is not None:
                v = jnp.where(v > thresh, v, jnp.inf)
            if a1 is None:
                a1, a2 = v, jnp.full_like(v, jnp.inf)
            else:
                t = jnp.maximum(a1, v)
                a1 = jnp.minimum(a1, v)
                a2 = jnp.minimum(a2, t)
        cat = jnp.concatenate([a1, a2], axis=1)
        t1 = jnp.min(cat, axis=1, keepdims=True)
        t2 = jnp.min(jnp.where(cat > t1, cat, jnp.inf), axis=1,
                     keepdims=True)
        return t2

    mn = _two_smallest(m, None)
    for _ in range(k // 2 - 1):
        mn = _two_smallest(m, mn)
    maskf = (m <= mn).astype(jnp.float32)
    summed = lax.dot_general(maskf, xn_all, (((1,), (0,)), ((), ())),
                             preferred_element_type=jnp.float32)
    dx = summed - xn_blk
    hpre = jnp.maximum(
        lax.dot_general(dx, wlut_ref[...], (((1,), (0,)), ((), ())),
                        preferred_element_type=jnp.float32) + blu_ref[...],
        0.0)
    hpre_ref[0] = hpre

    @pl.when(jnp.logical_and(b == 0, nb == 0))
    def _():
        ssum2_ref[...] = jnp.zeros_like(ssum2_ref)
        ssq2_ref[...] = jnp.zeros_like(ssq2_ref)

    ssum2_ref[...] += jnp.sum(hpre, axis=0, keepdims=True)
    ssq2_ref[...] += jnp.sum(hpre * hpre, axis=0, keepdims=True)


def _final_kernel(count, xpre_ref, hpre_ref, ssum_ref, ssq_ref, g1_ref,
                  b1_ref, ssum2_ref, ssq2_ref, g2_ref, b2_ref, out_ref):
    mu1 = ssum_ref[...] / count
    var1 = ssq_ref[...] / count - mu1 * mu1
    inv1 = lax.rsqrt(var1 + 1e-5) * g1_ref[...]
    sh1 = b1_ref[...] - mu1 * inv1
    x = jnp.maximum(xpre_ref[0] * inv1 + sh1, 0.0)
    mu2 = ssum2_ref[...] / count
    var2 = ssq2_ref[...] / count - mu2 * mu2
    inv2 = lax.rsqrt(var2 + 1e-5) * g2_ref[...]
    sh2 = b2_ref[...] - mu2 * inv2
    h = hpre_ref[0] * inv2 + sh2
    out_ref[0] = jnp.transpose(x + h)


def kernel(xyz1, xyz2, points1, points2, W_fuse, b_fuse, gamma1, beta1,
           W_lu, b_lu, gamma2, beta2):
    B, N, _ = xyz1.shape
    S = xyz2.shape[1]
    C1 = points1.shape[1]
    C2 = points2.shape[1]
    Co = W_fuse.shape[0]
    bn = _BN
    nb_total = N // bn
    count = float(B * N)

    p1t = jnp.transpose(points1, (0, 2, 1))
    p2t = jnp.transpose(points2, (0, 2, 1))
    w1t = jnp.transpose(W_fuse[:, :C1])
    w2t = jnp.transpose(W_fuse[:, C1:])
    wlut = jnp.transpose(W_lu)
    row = lambda v: jnp.reshape(v, (1, -1))

    grid = (B, nb_total)
    stat_spec = pl.BlockSpec((1, Co), lambda b, n: (0, 0))
    full_spec = lambda r, c: pl.BlockSpec((1, r, c), lambda b, n: (b, 0, 0))
    blk_spec = lambda c: pl.BlockSpec((1, bn, c), lambda b, n: (b, n, 0))
    mat_spec = lambda r, c: pl.BlockSpec((r, c), lambda b, n: (0, 0))

    xpre, ssum, ssq = pl.pallas_call(
        functools.partial(_fuse_kernel, nb_total, count),
        grid=grid,
        in_specs=[blk_spec(3), full_spec(S, 3), blk_spec(C1), full_spec(S, C2),
                  mat_spec(C1, Co), mat_spec(C2, Co), stat_spec],
        out_specs=[blk_spec(Co), stat_spec, stat_spec],
        out_shape=[jax.ShapeDtypeStruct((B, N, Co), jnp.float32),
                   jax.ShapeDtypeStruct((1, Co), jnp.float32),
                   jax.ShapeDtypeStruct((1, Co), jnp.float32)],
        interpret=_INTERPRET,
    )(xyz1, xyz2, p1t, p2t, w1t, w2t, row(b_fuse))

    hpre, ssum2, ssq2 = pl.pallas_call(
        functools.partial(_lap_kernel, count, 16),
        grid=grid,
        in_specs=[blk_spec(3), full_spec(N, 3), full_spec(N, Co), blk_spec(Co),
                  stat_spec, stat_spec, stat_spec, stat_spec,
                  mat_spec(Co, Co), stat_spec],
        out_specs=[blk_spec(Co), stat_spec, stat_spec],
        out_shape=[jax.ShapeDtypeStruct((B, N, Co), jnp.float32),
                   jax.ShapeDtypeStruct((1, Co), jnp.float32),
                   jax.ShapeDtypeStruct((1, Co), jnp.float32)],
        interpret=_INTERPRET,
    )(xyz1, xyz1, xpre, xpre, ssum, ssq, row(gamma1), row(beta1),
      wlut, row(b_lu))

    out = pl.pallas_call(
        functools.partial(_final_kernel, count),
        grid=grid,
        in_specs=[blk_spec(Co), blk_spec(Co), stat_spec, stat_spec,
                  stat_spec, stat_spec, stat_spec, stat_spec,
                  stat_spec, stat_spec],
        out_specs=pl.BlockSpec((1, Co, bn), lambda b, n: (b, 0, n)),
        out_shape=jax.ShapeDtypeStruct((B, Co, N), jnp.float32),
        interpret=_INTERPRET,
    )(xpre, hpre, ssum, ssq, row(gamma1), row(beta1),
      ssum2, ssq2, row(gamma2), row(beta2))
    return out
